# baseline (speedup 1.0000x reference)
import jax
import jax.numpy as jnp
from jax import lax
from jax.experimental import pallas as pl
from jax.experimental.pallas import tpu as pltpu
from jax.experimental.pallas import tpu_sc as plsc

BATCH = 16384
DIM = 32
VOCAB = 1_000_000

_info = plsc.get_sparse_core_info()
NC, NS, NL = _info.num_cores, _info.num_subcores, _info.num_lanes
NW = NC * NS
BPW = BATCH // NW
RND = 32
NJT = RND // 8
NROUND = BPW // RND


def _body(uid_hbm, iid_hbm, ut_hbm, it_hbm, out_hbm,
          uid_v, iid_v, ubuf_a, ibuf_a, ubuf_b, ibuf_b, out_v, drain_v,
          sem_a, sem_b):
    wid = lax.axis_index("s") * NC + lax.axis_index("c")
    base = wid * BPW

    pltpu.sync_copy(uid_hbm.at[pl.ds(base, BPW)], uid_v)
    pltpu.sync_copy(iid_hbm.at[pl.ds(base, BPW)], iid_v)

    iota = lax.iota(jnp.int32, NL)
    kk16 = (iota & 7) * 16
    jt_lo = iota >> 3

    def fire(g, ubuf, ibuf, sem):
        def fk(k, carry):
            sl = pl.ds(g * RND + (k // NL) * NL, NL)
            uvec = uid_v[sl]
            ivec = iid_v[sl]
            for vec, tbl, buf in ((uvec, ut_hbm, ubuf), (ivec, it_hbm, ibuf)):
                rv = jnp.take_along_axis(
                    vec, jnp.broadcast_to(k & (NL - 1), (NL,)).astype(jnp.int32),
                    axis=0)
                r = rv[0]
                off = (r // 16) * 16
                pltpu.async_copy(
                    tbl.at[:, :, pl.ds(off, 16)],
                    buf.at[:, k // 8, :, pl.ds((k % 8) * 16, 16)],
                    sem)
            return carry

        lax.fori_loop(0, RND, fk, 0)

    def consume(g, ubuf, ibuf, sem):
        pltpu.make_async_copy(
            ut_hbm.at[:, :, pl.ds(0, 2 * RND * 16)], drain_v, sem).wait()
        for grp in range(RND // NL):
            sl = pl.ds(g * RND + grp * NL, NL)
            lane_u = kk16 + (uid_v[sl] & 15)
            lane_i = kk16 + (iid_v[sl] & 15)
            jt = grp * (NL // 8) + jt_lo
            acc = jnp.zeros((NL,), jnp.float32)
            for d in range(DIM):
                iv = jnp.full((NL,), d // 8, jnp.int32)
                sv = jnp.full((NL,), d % 8, jnp.int32)
                u = plsc.load_gather(ubuf, [iv, jt, sv, lane_u])
                v = plsc.load_gather(ibuf, [iv, jt, sv, lane_i])
                acc = acc + u * v
            out_v[sl] = acc

    fire(0, ubuf_a, ibuf_a, sem_a)

    def step(t, carry):
        fire(2 * t + 1, ubuf_b, ibuf_b, sem_b)
        consume(2 * t, ubuf_a, ibuf_a, sem_a)

        @pl.when(t < NROUND // 2 - 1)
        def _():
            fire(2 * t + 2, ubuf_a, ibuf_a, sem_a)

        consume(2 * t + 1, ubuf_b, ibuf_b, sem_b)
        return carry

    lax.fori_loop(0, NROUND // 2, step, 0)

    pltpu.sync_copy(out_v, out_hbm.at[pl.ds(base, BPW)])


def kernel(user_ids, item_ids, user_table, item_table):
    mesh = plsc.VectorSubcoreMesh(core_axis_name="c", subcore_axis_name="s")
    f = pl.kernel(
        _body,
        mesh=mesh,
        out_type=jax.ShapeDtypeStruct((BATCH,), jnp.float32),
        scratch_types=[
            pltpu.VMEM((BPW,), jnp.int32),
            pltpu.VMEM((BPW,), jnp.int32),
            pltpu.VMEM((4, NJT, 8, 128), jnp.float32),
            pltpu.VMEM((4, NJT, 8, 128), jnp.float32),
            pltpu.VMEM((4, NJT, 8, 128), jnp.float32),
            pltpu.VMEM((4, NJT, 8, 128), jnp.float32),
            pltpu.VMEM((BPW,), jnp.float32),
            pltpu.VMEM((4, 8, 2 * RND * 16), jnp.float32),
            pltpu.SemaphoreType.DMA,
            pltpu.SemaphoreType.DMA,
        ],
        compiler_params=pltpu.CompilerParams(
            needs_layout_passes=False, use_tc_tiling_on_sc=True),
    )
    ut3 = user_table.T.reshape(4, 8, VOCAB)
    it3 = item_table.T.reshape(4, 8, VOCAB)
    return f(user_ids.astype(jnp.int32), item_ids.astype(jnp.int32), ut3, it3)


# speedup vs baseline: 1.0772x; 1.0772x over previous
import jax
import jax.numpy as jnp
from jax import lax
from jax.experimental import pallas as pl
from jax.experimental.pallas import tpu as pltpu
from jax.experimental.pallas import tpu_sc as plsc

BATCH = 16384
DIM = 32
VOCAB = 1_000_000

_info = plsc.get_sparse_core_info()
NC, NS, NL = _info.num_cores, _info.num_subcores, _info.num_lanes
NW = NC * NS
BPW = BATCH // NW
RND = 16
NJT = RND // 8
NROUND = BPW // RND


def _body(uid_hbm, iid_hbm, ut_hbm, it_hbm, out_hbm,
          uid_v, iid_v, ubuf_a, ibuf_a, ubuf_b, ibuf_b,
          ubuf_c, ibuf_c, ubuf_d, ibuf_d, out_v, drain_v,
          sem_a, sem_b, sem_c, sem_d):
    wid = lax.axis_index("s") * NC + lax.axis_index("c")
    base = wid * BPW

    pltpu.sync_copy(uid_hbm.at[pl.ds(base, BPW)], uid_v)
    pltpu.sync_copy(iid_hbm.at[pl.ds(base, BPW)], iid_v)

    iota = lax.iota(jnp.int32, NL)
    kk16 = (iota & 7) * 16
    jt_lo = iota >> 3

    def fire(g, ubuf, ibuf, sem):
        sl = pl.ds(g * RND, NL)
        uvec = uid_v[sl]
        ivec = iid_v[sl]

        def fk(k, carry):
            kvec = jnp.broadcast_to(k, (NL,)).astype(jnp.int32)
            for vec, tbl, buf in ((uvec, ut_hbm, ubuf), (ivec, it_hbm, ibuf)):
                r = jnp.take_along_axis(vec, kvec, axis=0)[0]
                off = (r // 16) * 16
                pltpu.async_copy(
                    tbl.at[:, :, pl.ds(off, 16)],
                    buf.at[:, k // 8, :, pl.ds((k % 8) * 16, 16)],
                    sem)
            return carry

        lax.fori_loop(0, RND, fk, 0)

    def consume(g, ubuf, ibuf, sem):
        pltpu.make_async_copy(
            ut_hbm.at[:, :, pl.ds(0, 2 * RND * 16)], drain_v, sem).wait()
        sl = pl.ds(g * RND, NL)
        lane_u = kk16 + (uid_v[sl] & 15)
        lane_i = kk16 + (iid_v[sl] & 15)

        acc = jnp.zeros((NL,), jnp.float32)
        for d in range(DIM):
            iv = jnp.full((NL,), d // 8, jnp.int32)
            sv = jnp.full((NL,), d % 8, jnp.int32)
            u = plsc.load_gather(ubuf, [iv, jt_lo, sv, lane_u])
            v = plsc.load_gather(ibuf, [iv, jt_lo, sv, lane_i])
            acc = acc + u * v
        out_v[sl] = acc

    ring = ((ubuf_a, ibuf_a, sem_a), (ubuf_b, ibuf_b, sem_b),
            (ubuf_c, ibuf_c, sem_c), (ubuf_d, ibuf_d, sem_d))
    for j in range(3):
        fire(j, *ring[j])

    def step(t, carry):
        g = 4 * t
        for j in range(4):
            nxt = g + j + 3
            nb = ring[(j + 3) % 4]

            @pl.when(nxt < NROUND)
            def _(nxt=nxt, nb=nb):
                fire(nxt, *nb)

            consume(g + j, *ring[j])
        return carry

    lax.fori_loop(0, NROUND // 4, step, 0)

    pltpu.sync_copy(out_v, out_hbm.at[pl.ds(base, BPW)])


def kernel(user_ids, item_ids, user_table, item_table):
    mesh = plsc.VectorSubcoreMesh(core_axis_name="c", subcore_axis_name="s")
    f = pl.kernel(
        _body,
        mesh=mesh,
        out_type=jax.ShapeDtypeStruct((BATCH,), jnp.float32),
        scratch_types=[
            pltpu.VMEM((BPW,), jnp.int32),
            pltpu.VMEM((BPW,), jnp.int32),
            pltpu.VMEM((4, NJT, 8, 128), jnp.float32),
            pltpu.VMEM((4, NJT, 8, 128), jnp.float32),
            pltpu.VMEM((4, NJT, 8, 128), jnp.float32),
            pltpu.VMEM((4, NJT, 8, 128), jnp.float32),
            pltpu.VMEM((4, NJT, 8, 128), jnp.float32),
            pltpu.VMEM((4, NJT, 8, 128), jnp.float32),
            pltpu.VMEM((4, NJT, 8, 128), jnp.float32),
            pltpu.VMEM((4, NJT, 8, 128), jnp.float32),
            pltpu.VMEM((BPW,), jnp.float32),
            pltpu.VMEM((4, 8, 2 * RND * 16), jnp.float32),
            pltpu.SemaphoreType.DMA,
            pltpu.SemaphoreType.DMA,
            pltpu.SemaphoreType.DMA,
            pltpu.SemaphoreType.DMA,
        ],
        compiler_params=pltpu.CompilerParams(
            needs_layout_passes=False, use_tc_tiling_on_sc=True),
    )
    ut3 = user_table.T.reshape(4, 8, VOCAB)
    it3 = item_table.T.reshape(4, 8, VOCAB)
    return f(user_ids.astype(jnp.int32), item_ids.astype(jnp.int32), ut3, it3)


# speedup vs baseline: 1.1645x; 1.0810x over previous
import jax
import jax.numpy as jnp
from jax import lax
from jax.experimental import pallas as pl
from jax.experimental.pallas import tpu as pltpu
from jax.experimental.pallas import tpu_sc as plsc

BATCH = 16384
DIM = 32
VOCAB = 1_000_000

_info = plsc.get_sparse_core_info()
NC, NS, NL = _info.num_cores, _info.num_subcores, _info.num_lanes
NW = NC * NS
BPW = BATCH // NW
RND = 16
NJT = RND // 8
NROUND = BPW // RND


def _body(uid_hbm, iid_hbm, ut_hbm, it_hbm, out_hbm,
          uid_v, iid_v, ubuf_a, ibuf_a, ubuf_b, ibuf_b, out_v, drain_v,
          sem_a, sem_b):
    wid = lax.axis_index("s") * NC + lax.axis_index("c")
    base = wid * BPW

    pltpu.sync_copy(uid_hbm.at[pl.ds(base, BPW)], uid_v)
    pltpu.sync_copy(iid_hbm.at[pl.ds(base, BPW)], iid_v)

    iota = lax.iota(jnp.int32, NL)
    kk16 = (iota & 7) * 16
    jt_lo = iota >> 3

    def fire(g, ubuf, ibuf, sem):
        sl = pl.ds(g * RND, NL)
        uvec = uid_v[sl]
        ivec = iid_v[sl]

        def fk(k, carry):
            kvec = jnp.broadcast_to(k, (NL,)).astype(jnp.int32)
            for vec, tbl, buf in ((uvec, ut_hbm, ubuf), (ivec, it_hbm, ibuf)):
                r = jnp.take_along_axis(vec, kvec, axis=0)[0]
                off = (r // 16) * 16
                pltpu.async_copy(
                    tbl.at[:, :, pl.ds(off, 16)],
                    buf.at[:, k // 8, :, pl.ds((k % 8) * 16, 16)],
                    sem)
            return carry

        lax.fori_loop(0, RND, fk, 0)

    def consume(g, ubuf, ibuf, sem):
        pltpu.make_async_copy(
            ut_hbm.at[:, :, pl.ds(0, 2 * RND * 16)], drain_v, sem).wait()
        sl = pl.ds(g * RND, NL)
        lane_u = kk16 + (uid_v[sl] & 15)
        lane_i = kk16 + (iid_v[sl] & 15)

        def dstep(dq, acc):
            iv = jnp.broadcast_to(dq.astype(jnp.int32), (NL,))
            for dr in range(8):
                sv = jnp.full((NL,), dr, jnp.int32)
                u = plsc.load_gather(ubuf, [iv, jt_lo, sv, lane_u])
                v = plsc.load_gather(ibuf, [iv, jt_lo, sv, lane_i])
                acc = acc + u * v
            return acc

        acc = lax.fori_loop(0, DIM // 8, dstep, jnp.zeros((NL,), jnp.float32))
        out_v[sl] = acc

    fire(0, ubuf_a, ibuf_a, sem_a)

    def step(t, carry):
        fire(2 * t + 1, ubuf_b, ibuf_b, sem_b)
        consume(2 * t, ubuf_a, ibuf_a, sem_a)

        @pl.when(t < NROUND // 2 - 1)
        def _():
            fire(2 * t + 2, ubuf_a, ibuf_a, sem_a)

        consume(2 * t + 1, ubuf_b, ibuf_b, sem_b)
        return carry

    lax.fori_loop(0, NROUND // 2, step, 0)

    pltpu.sync_copy(out_v, out_hbm.at[pl.ds(base, BPW)])


def kernel(user_ids, item_ids, user_table, item_table):
    mesh = plsc.VectorSubcoreMesh(core_axis_name="c", subcore_axis_name="s")
    f = pl.kernel(
        _body,
        mesh=mesh,
        out_type=jax.ShapeDtypeStruct((BATCH,), jnp.float32),
        scratch_types=[
            pltpu.VMEM((BPW,), jnp.int32),
            pltpu.VMEM((BPW,), jnp.int32),
            pltpu.VMEM((4, NJT, 8, 128), jnp.float32),
            pltpu.VMEM((4, NJT, 8, 128), jnp.float32),
            pltpu.VMEM((4, NJT, 8, 128), jnp.float32),
            pltpu.VMEM((4, NJT, 8, 128), jnp.float32),
            pltpu.VMEM((BPW,), jnp.float32),
            pltpu.VMEM((4, 8, 2 * RND * 16), jnp.float32),
            pltpu.SemaphoreType.DMA,
            pltpu.SemaphoreType.DMA,
        ],
        compiler_params=pltpu.CompilerParams(
            needs_layout_passes=False, use_tc_tiling_on_sc=True),
    )
    ut3 = user_table.T.reshape(4, 8, VOCAB)
    it3 = item_table.T.reshape(4, 8, VOCAB)
    return f(user_ids.astype(jnp.int32), item_ids.astype(jnp.int32), ut3, it3)
